# Initial kernel scaffold; baseline (speedup 1.0000x reference)
import jax
import jax.numpy as jnp
from jax.experimental import pallas as pl


def kernel(x, table):
    raise NotImplementedError("write your pallas kernel here")


# speedup vs baseline: 1.6577x; 1.6577x over previous
import functools
import math

import jax
import jax.numpy as jnp
from jax import lax
from jax.experimental import pallas as pl
from jax.experimental.pallas import tpu as pltpu
from jax.experimental.pallas import tpu_sc as plsc

VOCAB = 30522
EMBED = 512
SCALE = math.sqrt(EMBED)


_SCALE_BLK = 1024


def _scale_body(t_ref, o_ref):
    o_ref[...] = t_ref[...] * SCALE


def _scale_table(table):
    grid = (pl.cdiv(VOCAB, _SCALE_BLK),)
    return pl.pallas_call(
        _scale_body,
        grid=grid,
        in_specs=[pl.BlockSpec((_SCALE_BLK, EMBED), lambda i: (i, 0))],
        out_specs=pl.BlockSpec((_SCALE_BLK, EMBED), lambda i: (i, 0)),
        out_shape=jax.ShapeDtypeStruct((VOCAB, EMBED), jnp.float32),
    )(table)


@functools.lru_cache(maxsize=None)
def _make_gather(B, D):
    info = plsc.get_sparse_core_info()
    NC, NS = info.num_cores, info.num_subcores
    NW = NC * NS
    assert B % NW == 0
    b_per_w = B // NW
    C = 64
    assert b_per_w % C == 0
    nchunks = b_per_w // C
    mesh = plsc.VectorSubcoreMesh(core_axis_name="c", subcore_axis_name="s")

    @functools.partial(
        pl.kernel,
        mesh=mesh,
        out_type=jax.ShapeDtypeStruct((B, D), jnp.float32),
        scratch_types=[
            pltpu.VMEM((b_per_w,), jnp.int32),
            pltpu.VMEM((C, D), jnp.float32),
            pltpu.SemaphoreType.DMA,
        ],
    )
    def gather(table_hbm, idx_hbm, out_hbm, idx_v, buf, sem):
        wid = lax.axis_index("s") * NC + lax.axis_index("c")
        base = wid * b_per_w
        pltpu.sync_copy(idx_hbm.at[pl.ds(base, b_per_w)], idx_v)

        def body(c, _):
            off = pl.multiple_of(c * C, C)
            pltpu.async_copy(
                table_hbm.at[idx_v.at[pl.ds(off, C)]], buf, sem
            ).wait()
            pltpu.sync_copy(buf, out_hbm.at[pl.ds(base + off, C)])
            return _

        lax.fori_loop(0, nchunks, body, 0)

    return gather


def kernel(x, table):
    scaled = _scale_table(table)
    flat_idx = x.reshape(-1)
    out = _make_gather(flat_idx.shape[0], EMBED)(scaled, flat_idx)
    return out.reshape(x.shape + (EMBED,))


# speedup vs baseline: 1.9512x; 1.1770x over previous
import functools
import math

import jax
import jax.numpy as jnp
from jax import lax
from jax.experimental import pallas as pl
from jax.experimental.pallas import tpu as pltpu
from jax.experimental.pallas import tpu_sc as plsc

VOCAB = 30522
EMBED = 512
SCALE = math.sqrt(EMBED)


_SCALE_BLK = 1024


def _scale_body(t_ref, o_ref):
    o_ref[...] = t_ref[...] * SCALE


def _scale_table(table):
    grid = (pl.cdiv(VOCAB, _SCALE_BLK),)
    return pl.pallas_call(
        _scale_body,
        grid=grid,
        in_specs=[pl.BlockSpec((_SCALE_BLK, EMBED), lambda i: (i, 0))],
        out_specs=pl.BlockSpec((_SCALE_BLK, EMBED), lambda i: (i, 0)),
        out_shape=jax.ShapeDtypeStruct((VOCAB, EMBED), jnp.float32),
    )(table)


@functools.lru_cache(maxsize=None)
def _make_gather(B, D):
    info = plsc.get_sparse_core_info()
    NC, NS = info.num_cores, info.num_subcores
    NW = NC * NS
    assert B % NW == 0
    b_per_w = B // NW
    C = 40
    NBUF = 5
    LEAD = 2
    assert b_per_w % C == 0
    nchunks = b_per_w // C
    assert nchunks % NBUF == 0
    nlaps = nchunks // NBUF
    mesh = plsc.VectorSubcoreMesh(core_axis_name="c", subcore_axis_name="s")

    @functools.partial(
        pl.kernel,
        mesh=mesh,
        out_type=jax.ShapeDtypeStruct((B, D), jnp.float32),
        scratch_types=[
            pltpu.VMEM((b_per_w,), jnp.int32),
            pltpu.VMEM((NBUF, C, D), jnp.float32),
        ]
        + [pltpu.SemaphoreType.DMA] * (2 * NBUF),
    )
    def gather(table_hbm, idx_hbm, out_hbm, idx_v, bufs, *sems):
        gs, ws = sems[:NBUF], sems[NBUF:]
        wid = lax.axis_index("s") * NC + lax.axis_index("c")
        base = wid * b_per_w
        pltpu.sync_copy(idx_hbm.at[pl.ds(base, b_per_w)], idx_v)

        def _g(c, s):
            off = pl.multiple_of(c * C, 8)
            return pltpu.make_async_copy(
                table_hbm.at[idx_v.at[pl.ds(off, C)]], bufs.at[s], gs[s]
            )

        def _w(c, s):
            off = pl.multiple_of(c * C, 8)
            return pltpu.make_async_copy(
                bufs.at[s], out_hbm.at[pl.ds(base + off, C)], ws[s]
            )

        def step(c, s, do_wait_w, do_issue_g):
            _g(c, s).wait()
            _w(c, s).start()
            if do_issue_g:
                s2 = (s + LEAD) % NBUF
                if do_wait_w:
                    _w(c + LEAD - NBUF, s2).wait()
                _g(c + LEAD, s2).start()

        for c in range(LEAD):
            _g(c, c % NBUF).start()

        for s in range(NBUF):
            step(s, s, do_wait_w=(s + LEAD >= NBUF), do_issue_g=True)

        def lap(k, carry):
            c0 = k * NBUF
            for s in range(NBUF):
                step(c0 + s, s, do_wait_w=True, do_issue_g=True)
            return carry

        lax.fori_loop(1, nlaps - 1, lap, 0)

        c0 = nchunks - NBUF
        for s in range(NBUF):
            c = c0 + s
            step(c, s, do_wait_w=True, do_issue_g=(c + LEAD < nchunks))

        for s in range(NBUF):
            _w(c0 + s, s).wait()

    return gather


def kernel(x, table):
    scaled = _scale_table(table)
    flat_idx = x.reshape(-1)
    out = _make_gather(flat_idx.shape[0], EMBED)(scaled, flat_idx)
    return out.reshape(x.shape + (EMBED,))
